# baseline (speedup 1.0000x reference)
import functools

import jax
import jax.numpy as jnp
from jax import lax
from jax.experimental import pallas as pl
from jax.experimental.pallas import tpu as pltpu
from jax.experimental.pallas import tpu_sc as plsc

B = 16384
D = 16
NF = 10
VOCAB = 1000000
MLP_IN = (NF + 1) * D

NC = 2
NS = 16
NW = NC * NS

N_IDX = B * NF
N_PER_W = N_IDX // NW
CHUNK = 128
N_CHUNKS = N_PER_W // CHUNK


N16 = N_IDX * D
N16_PER_W = N16 // NW
HALF = N16_PER_W // 2
N16_CHUNKS = HALF // CHUNK


def _sc_gather(embT_hbm, fcT_hbm, idx16_hbm, idx_hbm, emb_out, fc_out,
               idx16_v, dst_v, fcidx_v, fc_v, sem_e, sem_f):
    wid = lax.axis_index("s") * NC + lax.axis_index("c")
    base16 = wid * N16_PER_W
    base = wid * N_PER_W

    pltpu.sync_copy(idx_hbm.at[pl.ds(base, N_PER_W)], fcidx_v)

    def fc_fire(j, carry):
        sl = pl.ds(j * CHUNK, CHUNK)
        pltpu.make_async_copy(fcT_hbm.at[fcidx_v.at[sl]], fc_v.at[sl, :], sem_f).start()
        return carry

    lax.fori_loop(0, N_CHUNKS, fc_fire, 0)

    for p in range(2):
        pltpu.sync_copy(idx16_hbm.at[pl.ds(base16 + p * HALF, HALF)], idx16_v)

        def fire(j, carry):
            sl = pl.ds(j * CHUNK, CHUNK)
            pltpu.make_async_copy(embT_hbm.at[idx16_v.at[sl]], dst_v.at[sl], sem_e).start()
            return carry

        lax.fori_loop(0, N16_CHUNKS, fire, 0)

        def drain(j, carry):
            sl = pl.ds(j * CHUNK, CHUNK)
            pltpu.make_async_copy(embT_hbm.at[idx16_v.at[sl]], dst_v.at[sl], sem_e).wait()
            return carry

        lax.fori_loop(0, N16_CHUNKS, drain, 0)
        pltpu.sync_copy(dst_v, emb_out.at[pl.ds(base16 + p * HALF, HALF)])

    def fc_drain(j, carry):
        sl = pl.ds(j * CHUNK, CHUNK)
        pltpu.make_async_copy(fcT_hbm.at[fcidx_v.at[sl]], fc_v.at[sl, :], sem_f).wait()
        return carry

    lax.fori_loop(0, N_CHUNKS, fc_drain, 0)
    pltpu.sync_copy(fc_v, fc_out.at[pl.ds(base, N_PER_W), :])


@functools.cache
def _gather_call():
    return pl.kernel(
        _sc_gather,
        out_type=(
            jax.ShapeDtypeStruct((N16,), jnp.float32),
            jax.ShapeDtypeStruct((N_IDX, 1), jnp.float32),
        ),
        mesh=plsc.VectorSubcoreMesh(core_axis_name="c", subcore_axis_name="s"),
        scratch_types=[
            pltpu.VMEM((HALF,), jnp.int32),
            pltpu.VMEM((HALF,), jnp.float32),
            pltpu.VMEM((N_PER_W,), jnp.int32),
            pltpu.VMEM((N_PER_W, 1), jnp.float32),
            pltpu.SemaphoreType.DMA,
            pltpu.SemaphoreType.DMA,
        ],
        compiler_params=pltpu.CompilerParams(use_tc_tiling_on_sc=False),
    )


BB = 2048


def _tc_dense(emb_ref, fc_ref, genre_ref, bias_ref, wg_ref, w1_ref, b1_ref,
              w2_ref, b2_ref, w3_ref, b3_ref, out_ref):
    emb = emb_ref[...]
    genre = genre_ref[...]
    eg = jnp.dot(genre, wg_ref[...], preferred_element_type=jnp.float32)

    fields = [emb[:, f * D:(f + 1) * D] for f in range(NF)] + [eg]
    s = fields[0]
    sos = fields[0] * fields[0]
    for v in fields[1:]:
        s = s + v
        sos = sos + v * v
    fm2 = 0.5 * jnp.sum(s * s - sos, axis=1)

    fm1 = bias_ref[0] + jnp.sum(fc_ref[...], axis=1) + jnp.sum(eg, axis=1)

    h = jnp.concatenate([emb, eg], axis=1)
    h = jnp.dot(h, w1_ref[...], preferred_element_type=jnp.float32) + b1_ref[...]
    h = jnp.maximum(h, 0.0)
    h = jnp.dot(h, w2_ref[...], preferred_element_type=jnp.float32) + b2_ref[...]
    h = jnp.maximum(h, 0.0)
    mlp = jnp.dot(h, w3_ref[...], preferred_element_type=jnp.float32)[:, 0] + b3_ref[0]

    out_ref[...] = jax.nn.sigmoid(fm1 + fm2 + mlp)


@functools.cache
def _dense_call():
  return pl.pallas_call(
    _tc_dense,
    grid=(B // BB,),
    in_specs=[
        pl.BlockSpec((BB, NF * D), lambda i: (i, 0)),
        pl.BlockSpec((BB, NF), lambda i: (i, 0)),
        pl.BlockSpec((BB, 18), lambda i: (i, 0)),
        pl.BlockSpec(memory_space=pltpu.SMEM),
        pl.BlockSpec((18, D), lambda i: (0, 0)),
        pl.BlockSpec((MLP_IN, 128), lambda i: (0, 0)),
        pl.BlockSpec((128,), lambda i: (0,)),
        pl.BlockSpec((128, 64), lambda i: (0, 0)),
        pl.BlockSpec((64,), lambda i: (0,)),
        pl.BlockSpec((64, 1), lambda i: (0, 0)),
        pl.BlockSpec(memory_space=pltpu.SMEM),
    ],
    out_specs=pl.BlockSpec((BB,), lambda i: (i,)),
    out_shape=jax.ShapeDtypeStruct((B,), jnp.float32),
  )


def kernel(x, bias, fc_table, W_genre, emb_table, W1, b1, W2, b2, W3, b3):
    idx_flat = x[:, :NF].reshape(-1)
    genre = x[:, NF:].astype(jnp.float32)
    offs = jnp.arange(D, dtype=jnp.int32) * VOCAB
    idx16 = (idx_flat[:, None] + offs[None, :]).reshape(-1)
    embT_flat = emb_table.T.reshape(-1)
    emb_g, fc_g = _gather_call()(embT_flat, fc_table, idx16, idx_flat)
    emb2 = emb_g.reshape(B, NF * D)
    fc2 = fc_g.reshape(B, NF)
    return _dense_call()(emb2, fc2, genre, bias, W_genre, W1, b1, W2, b2, W3, b3)


# speedup vs baseline: 2.2680x; 2.2680x over previous
import functools

import jax
import jax.numpy as jnp
from jax import lax
from jax.experimental import pallas as pl
from jax.experimental.pallas import tpu as pltpu
from jax.experimental.pallas import tpu_sc as plsc

B = 16384
D = 16
NF = 10
VOCAB = 1000000
MLP_IN = (NF + 1) * D

NC = 2
NS = 16
NW = NC * NS

N_IDX = B * NF
N_PER_W = N_IDX // NW
CHUNK = 128
N_CHUNKS = N_PER_W // CHUNK

SLAB = 3200
NSLAB = VOCAB // SLAB
TAIL0 = NSLAB * SLAB
TAIL = VOCAB - TAIL0
SLABS_PER_W = -(-NSLAB // NW)


def _sc_linearize(embT_hbm, out_hbm, slab_v, stage_v):
    wid = lax.axis_index("s") * NC + lax.axis_index("c")
    dvec = lax.iota(jnp.int32, 16)

    def transpose_rows(n):
        def trans(i, c):
            ivec = jnp.full((16,), 0, jnp.int32) + i
            r = plsc.load_gather(slab_v, [dvec, ivec])
            stage_v[pl.ds(i * D, D)] = r
            return c

        lax.fori_loop(0, n, trans, 0)

    def do_slab(j, carry):
        sid = wid + j * NW

        @pl.when(sid < NSLAB)
        def _():
            i0 = sid * SLAB
            pltpu.sync_copy(embT_hbm.at[:, pl.ds(i0, SLAB)], slab_v)
            transpose_rows(SLAB)
            pltpu.sync_copy(stage_v, out_hbm.at[pl.ds(i0 * D, SLAB * D)])

        return carry

    lax.fori_loop(0, SLABS_PER_W, do_slab, 0)

    @pl.when(wid < 12)
    def _():
        i0 = TAIL0 + wid * 128
        pltpu.sync_copy(embT_hbm.at[:, pl.ds(i0, 128)], slab_v.at[:, pl.ds(0, 128)])
        transpose_rows(128)
        pltpu.sync_copy(stage_v.at[pl.ds(0, 128 * D)],
                        out_hbm.at[pl.ds(i0 * D, 128 * D)])


@functools.cache
def _linearize_call():
    return pl.kernel(
        _sc_linearize,
        out_type=jax.ShapeDtypeStruct((VOCAB * D,), jnp.float32),
        mesh=plsc.VectorSubcoreMesh(core_axis_name="c", subcore_axis_name="s"),
        scratch_types=[
            pltpu.VMEM((D, SLAB), jnp.float32),
            pltpu.VMEM((SLAB * D,), jnp.float32),
        ],
        compiler_params=pltpu.CompilerParams(use_tc_tiling_on_sc=True,
                                             needs_layout_passes=False),
    )


def _sc_gather(emb_hbm, fc_hbm, idx_hbm, emb_out, fc_out,
               idx_v, emb_v, fc_v, sem_e, sem_f):
    wid = lax.axis_index("s") * NC + lax.axis_index("c")
    base = wid * N_PER_W
    pltpu.sync_copy(idx_hbm.at[pl.ds(base, N_PER_W)], idx_v)

    def fire(j, carry):
        sl = pl.ds(j * CHUNK, CHUNK)
        pltpu.make_async_copy(emb_hbm.at[idx_v.at[sl]], emb_v.at[sl], sem_e).start()
        pltpu.make_async_copy(fc_hbm.at[idx_v.at[sl]], fc_v.at[sl, :], sem_f).start()
        return carry

    lax.fori_loop(0, N_CHUNKS, fire, 0)

    def drain(j, carry):
        sl = pl.ds(j * CHUNK, CHUNK)
        pltpu.make_async_copy(emb_hbm.at[idx_v.at[sl]], emb_v.at[sl], sem_e).wait()
        pltpu.make_async_copy(fc_hbm.at[idx_v.at[sl]], fc_v.at[sl, :], sem_f).wait()
        return carry

    lax.fori_loop(0, N_CHUNKS, drain, 0)

    pltpu.sync_copy(emb_v, emb_out.at[pl.ds(base, N_PER_W)])
    pltpu.sync_copy(fc_v, fc_out.at[pl.ds(base, N_PER_W), :])


@functools.cache
def _gather_call():
    return pl.kernel(
        _sc_gather,
        out_type=(
            jax.ShapeDtypeStruct((N_IDX, D), jnp.float32),
            jax.ShapeDtypeStruct((N_IDX, 1), jnp.float32),
        ),
        mesh=plsc.VectorSubcoreMesh(core_axis_name="c", subcore_axis_name="s"),
        scratch_types=[
            pltpu.VMEM((N_PER_W,), jnp.int32),
            pltpu.VMEM((N_PER_W, D), jnp.float32),
            pltpu.VMEM((N_PER_W, 1), jnp.float32),
            pltpu.SemaphoreType.DMA,
            pltpu.SemaphoreType.DMA,
        ],
        compiler_params=pltpu.CompilerParams(use_tc_tiling_on_sc=False),
    )


BB = 2048


def _tc_dense(emb_ref, fc_ref, genre_ref, bias_ref, wg_ref, w1_ref, b1_ref,
              w2_ref, b2_ref, w3_ref, b3_ref, out_ref):
    emb = emb_ref[...]
    genre = genre_ref[...]
    eg = jnp.dot(genre, wg_ref[...], preferred_element_type=jnp.float32)

    fields = [emb[:, f * D:(f + 1) * D] for f in range(NF)] + [eg]
    s = fields[0]
    sos = fields[0] * fields[0]
    for v in fields[1:]:
        s = s + v
        sos = sos + v * v
    fm2 = 0.5 * jnp.sum(s * s - sos, axis=1)

    fm1 = bias_ref[0] + jnp.sum(fc_ref[...], axis=1) + jnp.sum(eg, axis=1)

    h = jnp.concatenate([emb, eg], axis=1)
    h = jnp.dot(h, w1_ref[...], preferred_element_type=jnp.float32) + b1_ref[...]
    h = jnp.maximum(h, 0.0)
    h = jnp.dot(h, w2_ref[...], preferred_element_type=jnp.float32) + b2_ref[...]
    h = jnp.maximum(h, 0.0)
    mlp = jnp.dot(h, w3_ref[...], preferred_element_type=jnp.float32)[:, 0] + b3_ref[0]

    out_ref[...] = jax.nn.sigmoid(fm1 + fm2 + mlp)


@functools.cache
def _dense_call():
  return pl.pallas_call(
    _tc_dense,
    grid=(B // BB,),
    in_specs=[
        pl.BlockSpec((BB, NF * D), lambda i: (i, 0)),
        pl.BlockSpec((BB, NF), lambda i: (i, 0)),
        pl.BlockSpec((BB, 18), lambda i: (i, 0)),
        pl.BlockSpec(memory_space=pltpu.SMEM),
        pl.BlockSpec((18, D), lambda i: (0, 0)),
        pl.BlockSpec((MLP_IN, 128), lambda i: (0, 0)),
        pl.BlockSpec((128,), lambda i: (0,)),
        pl.BlockSpec((128, 64), lambda i: (0, 0)),
        pl.BlockSpec((64,), lambda i: (0,)),
        pl.BlockSpec((64, 1), lambda i: (0, 0)),
        pl.BlockSpec(memory_space=pltpu.SMEM),
    ],
    out_specs=pl.BlockSpec((BB,), lambda i: (i,)),
    out_shape=jax.ShapeDtypeStruct((B,), jnp.float32),
  )


def kernel(x, bias, fc_table, W_genre, emb_table, W1, b1, W2, b2, W3, b3):
    idx_flat = x[:, :NF].reshape(-1)
    genre = x[:, NF:].astype(jnp.float32)
    emb_lin = _linearize_call()(emb_table.T)
    tail = emb_table[TAIL0 + 12 * 128:, :].reshape(-1)
    emb_lin = lax.dynamic_update_slice(emb_lin, tail, ((TAIL0 + 12 * 128) * D,))
    emb_lin = emb_lin.reshape(VOCAB, D)
    emb_g, fc_g = _gather_call()(emb_lin, fc_table, idx_flat)
    emb2 = emb_g.reshape(B, NF * D)
    fc2 = fc_g.reshape(B, NF)
    return _dense_call()(emb2, fc2, genre, bias, W_genre, W1, b1, W2, b2, W3, b3)


# speedup vs baseline: 5.0196x; 2.2133x over previous
import functools

import jax
import jax.numpy as jnp
from jax import lax
from jax.experimental import pallas as pl
from jax.experimental.pallas import tpu as pltpu
from jax.experimental.pallas import tpu_sc as plsc

B = 16384
D = 16
NF = 10
MLP_IN = (NF + 1) * D

NC = 2
NS = 16
NW = NC * NS

N_IDX = B * NF
N_PER_W = N_IDX // NW
CHUNK = 128
N_CHUNKS = N_PER_W // CHUNK


def _sc_gather(emb_hbm, fc_hbm, idx_hbm, emb_out, fc_out,
               idx_v, emb_v, fc_v, sem_e, sem_f):
    wid = lax.axis_index("s") * NC + lax.axis_index("c")
    base = wid * N_PER_W
    pltpu.sync_copy(idx_hbm.at[pl.ds(base, N_PER_W)], idx_v)

    def fire(j, carry):
        sl = pl.ds(j * CHUNK, CHUNK)
        pltpu.make_async_copy(emb_hbm.at[idx_v.at[sl]], emb_v.at[sl], sem_e).start()
        pltpu.make_async_copy(fc_hbm.at[idx_v.at[sl]], fc_v.at[sl], sem_f).start()
        return carry

    lax.fori_loop(0, N_CHUNKS, fire, 0)

    def drain(j, carry):
        sl = pl.ds(j * CHUNK, CHUNK)
        pltpu.make_async_copy(emb_hbm.at[idx_v.at[sl]], emb_v.at[sl], sem_e).wait()
        pltpu.make_async_copy(fc_hbm.at[idx_v.at[sl]], fc_v.at[sl], sem_f).wait()
        return carry

    lax.fori_loop(0, N_CHUNKS, drain, 0)

    pltpu.sync_copy(emb_v, emb_out.at[pl.ds(base, N_PER_W)])
    pltpu.sync_copy(fc_v, fc_out.at[pl.ds(base, N_PER_W)])


@functools.cache
def _gather_call():
    return pl.kernel(
        _sc_gather,
        out_type=(
            jax.ShapeDtypeStruct((N_IDX, D), jnp.float32),
            jax.ShapeDtypeStruct((N_IDX,), jnp.float32),
        ),
        mesh=plsc.VectorSubcoreMesh(core_axis_name="c", subcore_axis_name="s"),
        scratch_types=[
            pltpu.VMEM((N_PER_W,), jnp.int32),
            pltpu.VMEM((N_PER_W, D), jnp.float32),
            pltpu.VMEM((N_PER_W,), jnp.float32),
            pltpu.SemaphoreType.DMA,
            pltpu.SemaphoreType.DMA,
        ],
        compiler_params=pltpu.CompilerParams(use_tc_tiling_on_sc=False),
    )


VOCAB = 1000000
TSLAB = 3200
NTS = VOCAB // TSLAB
TTAIL0 = NTS * TSLAB


def _tc_transpose(src_ref, out_ref):
    x = src_ref[...]
    z = x.T
    y = z.reshape(TSLAB // 8, 8, D)
    out_ref[...] = jnp.concatenate([y[:, k, :] for k in range(8)], axis=1)


@functools.cache
def _transpose_call():
    return pl.pallas_call(
        _tc_transpose,
        grid=(NTS,),
        in_specs=[pl.BlockSpec((D, TSLAB), lambda i: (0, i))],
        out_specs=pl.BlockSpec((TSLAB // 8, 128), lambda i: (i, 0)),
        out_shape=jax.ShapeDtypeStruct((VOCAB * D // 128, 128), jnp.float32),
    )


BB = 2048


def _tc_dense(emb_ref, fc_ref, genre_ref, bias_ref, wg_ref, w1_ref, b1_ref,
              w2_ref, b2_ref, w3_ref, b3_ref, out_ref):
    emb = emb_ref[...]
    genre = genre_ref[...]
    eg = jnp.dot(genre, wg_ref[...], preferred_element_type=jnp.float32)

    fields = [emb[:, f * D:(f + 1) * D] for f in range(NF)] + [eg]
    s = fields[0]
    sos = fields[0] * fields[0]
    for v in fields[1:]:
        s = s + v
        sos = sos + v * v
    fm2 = 0.5 * jnp.sum(s * s - sos, axis=1)

    fm1 = bias_ref[0] + jnp.sum(fc_ref[...], axis=1) + jnp.sum(eg, axis=1)

    h = jnp.concatenate([emb, eg], axis=1)
    h = jnp.dot(h, w1_ref[...], preferred_element_type=jnp.float32) + b1_ref[...]
    h = jnp.maximum(h, 0.0)
    h = jnp.dot(h, w2_ref[...], preferred_element_type=jnp.float32) + b2_ref[...]
    h = jnp.maximum(h, 0.0)
    mlp = jnp.dot(h, w3_ref[...], preferred_element_type=jnp.float32)[:, 0] + b3_ref[0]

    out_ref[...] = jax.nn.sigmoid(fm1 + fm2 + mlp)


@functools.cache
def _dense_call():
  return pl.pallas_call(
    _tc_dense,
    grid=(B // BB,),
    in_specs=[
        pl.BlockSpec((BB, NF * D), lambda i: (i, 0)),
        pl.BlockSpec((BB, NF), lambda i: (i, 0)),
        pl.BlockSpec((BB, 18), lambda i: (i, 0)),
        pl.BlockSpec(memory_space=pltpu.SMEM),
        pl.BlockSpec((18, D), lambda i: (0, 0)),
        pl.BlockSpec((MLP_IN, 128), lambda i: (0, 0)),
        pl.BlockSpec((128,), lambda i: (0,)),
        pl.BlockSpec((128, 64), lambda i: (0, 0)),
        pl.BlockSpec((64,), lambda i: (0,)),
        pl.BlockSpec((64, 1), lambda i: (0, 0)),
        pl.BlockSpec(memory_space=pltpu.SMEM),
    ],
    out_specs=pl.BlockSpec((BB,), lambda i: (i,)),
    out_shape=jax.ShapeDtypeStruct((B,), jnp.float32),
  )


def kernel(x, bias, fc_table, W_genre, emb_table, W1, b1, W2, b2, W3, b3):
    idx_flat = x[:, :NF].reshape(-1)
    genre = x[:, NF:].astype(jnp.float32)
    emb_lin = _transpose_call()(emb_table.T)
    tail = emb_table[TTAIL0:, :].reshape(TSLAB // 16, 128)
    emb_lin = lax.dynamic_update_slice(emb_lin, tail, (TTAIL0 * D // 128, 0))
    emb_lin = emb_lin.reshape(VOCAB, D)
    emb_g, fc_g = _gather_call()(emb_lin, fc_table.reshape(-1), idx_flat)
    emb2 = emb_g.reshape(B, NF * D)
    fc2 = fc_g.reshape(B, NF)
    return _dense_call()(emb2, fc2, genre, bias, W_genre, W1, b1, W2, b2, W3, b3)
